# Initial kernel scaffold; baseline (speedup 1.0000x reference)
import jax
import jax.numpy as jnp
from jax.experimental import pallas as pl


def kernel(src, x_mark_enc, x_mark_dec, params):
    raise NotImplementedError("write your pallas kernel here")


# speedup vs baseline: 1.2994x; 1.2994x over previous
import functools
import math

import numpy as np
import jax
import jax.numpy as jnp
from jax.experimental import pallas as pl

D_MODEL = 512
N_HEADS = 8
DH = 64
D_FF = 2048
FACTOR = 5
C_IN = 7
C_OUT = 24
SEQ_LEN = 96
LABEL_LEN = 48
PRED_LEN = 24

_F32 = jnp.float32
_BF16 = jnp.bfloat16
_HI = jax.lax.Precision.HIGHEST


def _sincos_np(n_pos, d):
    pe = np.zeros((n_pos, d), dtype=np.float32)
    position = np.arange(n_pos, dtype=np.float32)[:, None]
    div_term = np.exp(np.arange(0, d, 2, dtype=np.float32) * -(math.log(10000.0) / d))
    pe[:, 0::2] = np.sin(position * div_term)
    pe[:, 1::2] = np.cos(position * div_term)
    return pe


_TABLE = np.zeros((80, D_MODEL), np.float32)
_TABLE[:76] = np.concatenate(
    [_sincos_np(13, D_MODEL), _sincos_np(32, D_MODEL),
     _sincos_np(7, D_MODEL), _sincos_np(24, D_MODEL)], axis=0)
_OFFS = np.array([0, 13, 45, 52], np.int32)
_POS = _sincos_np(SEQ_LEN, D_MODEL)

_samp_cache = {}


def _samp_consts(L, fold):
    ck = (L, fold)
    if ck not in _samp_cache:
        U_part = min(int(FACTOR * np.ceil(np.log(L))), L)
        u = min(int(FACTOR * np.ceil(np.log(L))), L)
        rng = jax.random.fold_in(jax.random.key(42), fold)
        idx = np.asarray(jax.random.randint(rng, (L, U_part), 0, L))
        _samp_cache[ck] = (idx, u)
    return _samp_cache[ck]


for _ck in ((96, 0), (48, 1), (24, 2), (72, 10), (72, 11)):
    _samp_consts(*_ck)


def _gelu(x):
    return x * 0.5 * (1.0 + jax.lax.erf(x * (1.0 / math.sqrt(2.0))))


def _lnv(y, g, b):
    m = jnp.mean(y, axis=-1, keepdims=True)
    v = jnp.mean((y - m) * (y - m), axis=-1, keepdims=True)
    return (y - m) / jnp.sqrt(v + 1e-5) * g + b


def _mm_body(x_ref, w_ref, b_ref, o_ref):
    o_ref[...] = jnp.dot(x_ref[...], w_ref[...], preferred_element_type=_F32) + b_ref[...]


def _mm(x, wT, b, br=512):
    N, Din = x.shape
    Dout = wT.shape[1]
    return pl.pallas_call(
        _mm_body,
        grid=(N // br,),
        in_specs=[
            pl.BlockSpec((br, Din), lambda i: (i, 0)),
            pl.BlockSpec((Din, Dout), lambda i: (0, 0)),
            pl.BlockSpec((1, Dout), lambda i: (0, 0)),
        ],
        out_specs=pl.BlockSpec((br, Dout), lambda i: (i, 0)),
        out_shape=jax.ShapeDtypeStruct((N, Dout), _F32),
    )(x, wT, jnp.reshape(b, (1, -1)))


def _oproj_body(a_ref, x_ref, w_ref, b_ref, g_ref, be_ref, o_ref):
    y = x_ref[...] + jnp.dot(a_ref[...], w_ref[...], preferred_element_type=_F32) + b_ref[...]
    o_ref[...] = _lnv(y, g_ref[...], be_ref[...])


def _oproj_ln(a, x, wT, b, ln, br=512):
    N = x.shape[0]
    return pl.pallas_call(
        _oproj_body,
        grid=(N // br,),
        in_specs=[
            pl.BlockSpec((br, D_MODEL), lambda i: (i, 0)),
            pl.BlockSpec((br, D_MODEL), lambda i: (i, 0)),
            pl.BlockSpec((D_MODEL, D_MODEL), lambda i: (0, 0)),
            pl.BlockSpec((1, D_MODEL), lambda i: (0, 0)),
            pl.BlockSpec((1, D_MODEL), lambda i: (0, 0)),
            pl.BlockSpec((1, D_MODEL), lambda i: (0, 0)),
        ],
        out_specs=pl.BlockSpec((br, D_MODEL), lambda i: (i, 0)),
        out_shape=jax.ShapeDtypeStruct((N, D_MODEL), _F32),
    )(a, x, wT, jnp.reshape(b, (1, -1)), jnp.reshape(ln["g"], (1, -1)),
      jnp.reshape(ln["b"], (1, -1)))


def _ffn_body(x_ref, w1_ref, b1_ref, w2_ref, b2_ref, g1_ref, be1_ref,
              g2_ref, be2_ref, o_ref, *, two):
    x = x_ref[...]
    h = _gelu(jnp.dot(x, w1_ref[...], preferred_element_type=_F32) + b1_ref[...])
    y = x + jnp.dot(h, w2_ref[...], preferred_element_type=_F32) + b2_ref[...]
    y = _lnv(y, g1_ref[...], be1_ref[...])
    if two:
        y = _lnv(y, g2_ref[...], be2_ref[...])
    o_ref[...] = y


def _ffn_ln(x, w1T, b1, w2T, b2, ln1, ln2, br=256):
    N = x.shape[0]
    two = ln2 is not None
    if ln2 is None:
        g2 = np.ones((1, D_MODEL), np.float32)
        be2 = np.zeros((1, D_MODEL), np.float32)
    else:
        g2 = jnp.reshape(ln2["g"], (1, -1))
        be2 = jnp.reshape(ln2["b"], (1, -1))
    return pl.pallas_call(
        functools.partial(_ffn_body, two=two),
        grid=(N // br,),
        in_specs=[
            pl.BlockSpec((br, D_MODEL), lambda i: (i, 0)),
            pl.BlockSpec((D_MODEL, D_FF), lambda i: (0, 0)),
            pl.BlockSpec((1, D_FF), lambda i: (0, 0)),
            pl.BlockSpec((D_FF, D_MODEL), lambda i: (0, 0)),
            pl.BlockSpec((1, D_MODEL), lambda i: (0, 0)),
            pl.BlockSpec((1, D_MODEL), lambda i: (0, 0)),
            pl.BlockSpec((1, D_MODEL), lambda i: (0, 0)),
            pl.BlockSpec((1, D_MODEL), lambda i: (0, 0)),
            pl.BlockSpec((1, D_MODEL), lambda i: (0, 0)),
        ],
        out_specs=pl.BlockSpec((br, D_MODEL), lambda i: (i, 0)),
        out_shape=jax.ShapeDtypeStruct((N, D_MODEL), _F32),
    )(x, w1T, jnp.reshape(b1, (1, -1)), w2T, jnp.reshape(b2, (1, -1)),
      jnp.reshape(ln1["g"], (1, -1)), jnp.reshape(ln1["b"], (1, -1)), g2, be2)


def _sampled_m(x, ap, L, fold):
    B = x.shape[0]
    q = (x @ ap["q"]["w"].T + ap["q"]["b"]).reshape(B, L, N_HEADS, DH)
    k = (x @ ap["k"]["w"].T + ap["k"]["b"]).reshape(B, L, N_HEADS, DH)
    Q = jnp.transpose(q, (0, 2, 1, 3))
    Kk = jnp.transpose(k, (0, 2, 1, 3))
    idx, u = _samp_consts(L, fold)
    Ks = Kk[:, :, idx, :]
    qs = jnp.einsum('bhld,bhlud->bhlu', Q, Ks)
    M = qs.max(-1) - qs.sum(-1) / L
    return M, u


def _attn_body(qkv_ref, m_ref, base_ref, o_ref, *, L, u, masked, bc):
    Z = qkv_ref[...]
    inv = 1.0 / math.sqrt(DH)
    row = jax.lax.broadcasted_iota(jnp.int32, (L, L), 0)
    col = jax.lax.broadcasted_iota(jnp.int32, (L, L), 1)
    for j in range(bc):
        for h in range(N_HEADS):
            q = Z[j, :, h * DH:(h + 1) * DH]
            k = Z[j, :, D_MODEL + h * DH:D_MODEL + (h + 1) * DH]
            v = Z[j, :, 2 * D_MODEL + h * DH:2 * D_MODEL + (h + 1) * DH]
            Mr = m_ref[j, h, :][None, :]
            Mc = jnp.transpose(Mr, (1, 0))
            Mb = jnp.broadcast_to(Mr, (L, L))
            rank = jnp.sum((Mb > Mc).astype(jnp.int32)
                           + ((Mb == Mc) & (col < row)).astype(jnp.int32), axis=1)
            sel = rank < u
            S = jax.lax.dot_general(q, k, (((1,), (1,)), ((), ())),
                                    preferred_element_type=_F32) * inv
            if masked:
                S = jnp.where(col > row, -1e9, S)
            S = S - jnp.max(S, axis=1, keepdims=True)
            e = jnp.exp(S)
            A = e / jnp.sum(e, axis=1, keepdims=True)
            out = jnp.dot(A, v, preferred_element_type=_F32)
            if masked:
                base = jnp.dot(base_ref[...], v, preferred_element_type=_F32,
                               precision=_HI)
            else:
                base = jnp.broadcast_to(jnp.mean(v, axis=0, keepdims=True), (L, DH))
            o_ref[j, :, h * DH:(h + 1) * DH] = jnp.where(sel[:, None], out, base)


def _prob_attn(qkv, L, fold, masked, x, ap, bc=4):
    B = qkv.shape[0]
    M, u = _sampled_m(x, ap, L, fold)
    bw = np.tril(np.ones((L, L), np.float32))
    return pl.pallas_call(
        functools.partial(_attn_body, L=L, u=u, masked=masked, bc=bc),
        grid=(B // bc,),
        in_specs=[
            pl.BlockSpec((bc, L, 3 * D_MODEL), lambda i: (i, 0, 0)),
            pl.BlockSpec((bc, N_HEADS, L), lambda i: (i, 0, 0)),
            pl.BlockSpec((L, L), lambda i: (0, 0)),
        ],
        out_specs=pl.BlockSpec((bc, L, D_MODEL), lambda i: (i, 0, 0)),
        out_shape=jax.ShapeDtypeStruct((B, L, D_MODEL), _F32),
    )(qkv, M, bw)


def _cross_body(q_ref, kv_ref, o_ref, *, Lq, Lk, bc):
    Zq = q_ref[...]
    Zkv = kv_ref[...]
    inv = 1.0 / math.sqrt(DH)
    for j in range(bc):
        for h in range(N_HEADS):
            q = Zq[j, :, h * DH:(h + 1) * DH]
            k = Zkv[j, :, h * DH:(h + 1) * DH]
            v = Zkv[j, :, D_MODEL + h * DH:D_MODEL + (h + 1) * DH]
            S = jax.lax.dot_general(q, k, (((1,), (1,)), ((), ())),
                                    preferred_element_type=_F32) * inv
            S = S - jnp.max(S, axis=1, keepdims=True)
            e = jnp.exp(S)
            A = e / jnp.sum(e, axis=1, keepdims=True)
            o_ref[j, :, h * DH:(h + 1) * DH] = jnp.dot(A, v, preferred_element_type=_F32)


def _cross_attn(q, kv, bc=4):
    B, Lq = q.shape[0], q.shape[1]
    Lk = kv.shape[1]
    return pl.pallas_call(
        functools.partial(_cross_body, Lq=Lq, Lk=Lk, bc=bc),
        grid=(B // bc,),
        in_specs=[
            pl.BlockSpec((bc, Lq, D_MODEL), lambda i: (i, 0, 0)),
            pl.BlockSpec((bc, Lk, 2 * D_MODEL), lambda i: (i, 0, 0)),
        ],
        out_specs=pl.BlockSpec((bc, Lq, D_MODEL), lambda i: (i, 0, 0)),
        out_shape=jax.ShapeDtypeStruct((B, Lq, D_MODEL), _F32),
    )(q, kv)


def _embed_body(xu_ref, idx_ref, wt_ref, tab_ref, pos_ref, o_ref, *, L, bc):
    xu = xu_ref[...].reshape(bc * L, 24).astype(_BF16).astype(_F32)
    wt = wt_ref[...].astype(_BF16).astype(_F32)
    y = jnp.dot(xu, wt, preferred_element_type=_F32, precision=_HI)
    ii = idx_ref[...].reshape(bc * L, 4)
    io = jax.lax.broadcasted_iota(jnp.int32, (bc * L, 80), 1)
    oh = ((ii[:, 0:1] == io).astype(_F32) + (ii[:, 1:2] == io).astype(_F32)
          + (ii[:, 2:3] == io).astype(_F32) + (ii[:, 3:4] == io).astype(_F32))
    y = y + jnp.dot(oh, tab_ref[...], preferred_element_type=_F32, precision=_HI)
    o_ref[...] = y.reshape(bc, L, D_MODEL) + pos_ref[...][None]


def _embed(x, x_mark, conv_w, bc=8):
    B, L = x.shape[0], x.shape[1]
    xu = jnp.concatenate(
        [jnp.roll(x, 1, axis=1), x, jnp.roll(x, -1, axis=1)], axis=-1)
    xu = jnp.pad(xu, ((0, 0), (0, 0), (0, 3)))
    wt = jnp.pad(jnp.transpose(conv_w, (2, 1, 0)).reshape(3 * C_IN, D_MODEL),
                 ((0, 3), (0, 0)))
    idx = x_mark.astype(jnp.int32) + _OFFS[None, None, :]
    pos = _POS[:L]
    return pl.pallas_call(
        functools.partial(_embed_body, L=L, bc=bc),
        grid=(B // bc,),
        in_specs=[
            pl.BlockSpec((bc, L, 24), lambda i: (i, 0, 0)),
            pl.BlockSpec((bc, L, 4), lambda i: (i, 0, 0)),
            pl.BlockSpec((24, D_MODEL), lambda i: (0, 0)),
            pl.BlockSpec((80, D_MODEL), lambda i: (0, 0)),
            pl.BlockSpec((L, D_MODEL), lambda i: (0, 0)),
        ],
        out_specs=pl.BlockSpec((bc, L, D_MODEL), lambda i: (i, 0, 0)),
        out_shape=jax.ShapeDtypeStruct((B, L, D_MODEL), _F32),
    )(xu, idx, wt, _TABLE, pos)


def _conv_body(x_ref, w0_ref, w1_ref, w2_ref, b_ref, bg_ref, bb_ref,
               bm_ref, bv_ref, o_ref, *, L, bc):
    x = x_ref[...]
    xm = jnp.concatenate([x[:, L - 1:L, :], x[:, :L - 1, :]], axis=1)
    xp = jnp.concatenate([x[:, 1:, :], x[:, 0:1, :]], axis=1)
    y = (jnp.dot(xm.reshape(bc * L, D_MODEL), w0_ref[...], preferred_element_type=_F32)
         + jnp.dot(x.reshape(bc * L, D_MODEL), w1_ref[...], preferred_element_type=_F32)
         + jnp.dot(xp.reshape(bc * L, D_MODEL), w2_ref[...], preferred_element_type=_F32)
         + b_ref[...])
    y = (y - bm_ref[...]) / jnp.sqrt(bv_ref[...] + 1e-5) * bg_ref[...] + bb_ref[...]
    y = jnp.where(y > 0, y, jnp.exp(jnp.minimum(y, 0.0)) - 1.0)
    y3 = y.reshape(bc, L // 2, 2, D_MODEL)
    a = y3[:, :, 0, :]
    b2 = y3[:, :, 1, :]
    prev = jnp.concatenate(
        [jnp.full((bc, 1, D_MODEL), -jnp.inf, _F32), b2[:, :L // 2 - 1, :]], axis=1)
    o_ref[...] = jnp.maximum(jnp.maximum(a, b2), prev)


def _conv_layer(p, x, bc=8):
    B, L = x.shape[0], x.shape[1]
    w = p["w"]
    return pl.pallas_call(
        functools.partial(_conv_body, L=L, bc=bc),
        grid=(B // bc,),
        in_specs=[pl.BlockSpec((bc, L, D_MODEL), lambda i: (i, 0, 0))]
        + [pl.BlockSpec((D_MODEL, D_MODEL), lambda i: (0, 0))] * 3
        + [pl.BlockSpec((1, D_MODEL), lambda i: (0, 0))] * 5,
        out_specs=pl.BlockSpec((bc, L // 2, D_MODEL), lambda i: (i, 0, 0)),
        out_shape=jax.ShapeDtypeStruct((B, L // 2, D_MODEL), _F32),
    )(x, w[:, :, 0].T, w[:, :, 1].T, w[:, :, 2].T,
      jnp.reshape(p["b"], (1, -1)), jnp.reshape(p["bn_g"], (1, -1)),
      jnp.reshape(p["bn_b"], (1, -1)), jnp.reshape(p["bn_m"], (1, -1)),
      jnp.reshape(p["bn_v"], (1, -1)))


def _qkv_cat(ap):
    wT = jnp.concatenate([ap["q"]["w"], ap["k"]["w"], ap["v"]["w"]], axis=0).T
    b = jnp.concatenate([ap["q"]["b"], ap["k"]["b"], ap["v"]["b"]], axis=0)
    return wT, b


def _enc_layer(p, x, fold, extra_ln):
    B, L = x.shape[0], x.shape[1]
    N = B * L
    wT, b = _qkv_cat(p["attn"])
    qkv = _mm(x.reshape(N, D_MODEL), wT, b).reshape(B, L, 3 * D_MODEL)
    ctx = _prob_attn(qkv, L, fold, False, x, p["attn"])
    x1 = _oproj_ln(ctx.reshape(N, D_MODEL), x.reshape(N, D_MODEL),
                   p["attn"]["o"]["w"].T, p["attn"]["o"]["b"], p["norm1"])
    y = _ffn_ln(x1, p["conv1_w"].T, p["conv1_b"], p["conv2_w"].T, p["conv2_b"],
                p["norm2"], extra_ln)
    return y.reshape(B, L, D_MODEL)


def _dec_layer(p, d, enc_out, fold, extra_ln):
    B, L = d.shape[0], d.shape[1]
    N = B * L
    Lk = enc_out.shape[1]
    wT, b = _qkv_cat(p["self"])
    qkv = _mm(d.reshape(N, D_MODEL), wT, b).reshape(B, L, 3 * D_MODEL)
    ctx = _prob_attn(qkv, L, fold, True, d, p["self"])
    mixed = ctx.reshape(B, L, N_HEADS, DH).transpose(0, 2, 1, 3).reshape(B, L, D_MODEL)
    x1 = _oproj_ln(mixed.reshape(N, D_MODEL), d.reshape(N, D_MODEL),
                   p["self"]["o"]["w"].T, p["self"]["o"]["b"], p["norm1"])
    q = _mm(x1, p["cross"]["q"]["w"].T, p["cross"]["q"]["b"])
    kvT = jnp.concatenate([p["cross"]["k"]["w"], p["cross"]["v"]["w"]], axis=0).T
    kvb = jnp.concatenate([p["cross"]["k"]["b"], p["cross"]["v"]["b"]], axis=0)
    kv = _mm(enc_out.reshape(B * Lk, D_MODEL), kvT, kvb).reshape(B, Lk, 2 * D_MODEL)
    cctx = _cross_attn(q.reshape(B, L, D_MODEL), kv)
    x2 = _oproj_ln(cctx.reshape(N, D_MODEL), x1,
                   p["cross"]["o"]["w"].T, p["cross"]["o"]["b"], p["norm2"])
    y = _ffn_ln(x2, p["conv1_w"].T, p["conv1_b"], p["conv2_w"].T, p["conv2_b"],
                p["norm3"], extra_ln)
    return y.reshape(B, L, D_MODEL)


def kernel(src, x_mark_enc, x_mark_dec, params):
    B = src.shape[0]
    p = params
    x = _embed(src, x_mark_enc, p["enc_emb_conv"])
    x = _enc_layer(p["enc"][0], x, 0, None)
    x = _conv_layer(p["enc_conv"][0], x)
    x = _enc_layer(p["enc"][1], x, 1, None)
    x = _conv_layer(p["enc_conv"][1], x)
    x = _enc_layer(p["enc"][2], x, 2, p["enc_norm"])
    dec_in = jnp.concatenate(
        [src[:, -LABEL_LEN:, :], jnp.zeros((B, PRED_LEN, C_IN), src.dtype)], axis=1)
    d = _embed(dec_in, x_mark_dec, p["dec_emb_conv"])
    d = _dec_layer(p["dec"][0], d, x, 10, None)
    d = _dec_layer(p["dec"][1], d, x, 11, p["dec_norm"])
    dl = d[:, -PRED_LEN:, :].reshape(B * PRED_LEN, D_MODEL)
    out = _mm(dl, p["proj"]["w"].T, p["proj"]["b"])
    return out.reshape(B, PRED_LEN, C_OUT)
